# baseline (speedup 1.0000x reference)
import jax
import jax.numpy as jnp
from jax import lax
from jax.experimental import pallas as pl
from jax.experimental.pallas import tpu as pltpu
from jax.experimental.pallas import tpu_sc as plsc

ROWS, N = 64, 8192
K = int(0.05 * N)
NC, NS, L = 2, 16, 16
NW = NC * NS
ROWS_PER_W = ROWS // NW
NVEC = N // L
INT_MIN = -2147483648

S1 = 10
S2 = 8
S3 = 32 - S1 - S2
PAD = 4 * L


def _ordered_bits_to_f32(cand_u):
    bits = jnp.where(cand_u < 0, cand_u ^ INT_MIN, ~cand_u)
    return lax.bitcast_convert_type(bits, jnp.float32)


def _body(x_hbm, out_hbm, x_v, cand_a, cand_b, sem):
    wid = lax.axis_index("s") * NC + lax.axis_index("c")
    base = wid * ROWS_PER_W
    pltpu.sync_copy(x_hbm.at[pl.ds(base, ROWS_PER_W)], x_v)

    UNROLL = 8
    ONE = jnp.full((L,), 1, jnp.int32)
    ZERO = jnp.full((L,), 0, jnp.int32)
    NEG_INF = jnp.full((L,), -jnp.inf, jnp.float32)

    def lane_sum(acc):
        parts = [acc[e] for e in range(L)]
        while len(parts) > 1:
            nxt = [parts[i] + parts[i + 1]
                   for i in range(0, len(parts) - 1, 2)]
            if len(parts) % 2:
                nxt.append(parts[-1])
            parts = nxt
        return parts[0]

    def popcnt(m):
        return plsc.all_reduce_population_count(m)[0]

    for r in range(ROWS_PER_W):
        def dbl_step(d, carry):
            prefix_u, cnt_p = carry
            b_hi = lax.shift_left(jnp.int32(1), 31 - 2 * d)
            b_lo = lax.shift_left(jnp.int32(1), 30 - 2 * d)
            c1 = prefix_u | b_hi
            c12 = c1 | b_lo
            c2 = prefix_u | b_lo
            f1 = _ordered_bits_to_f32(c1)
            f12 = _ordered_bits_to_f32(c12)
            f2 = _ordered_bits_to_f32(c2)

            def count(j, accs):
                a1, a12, a2 = accs
                for u in range(UNROLL):
                    xv = x_v[r, pl.ds((j * UNROLL + u) * L, L)]
                    a1 = a1 + jnp.where(xv >= f1, ONE, ZERO)
                    a12 = a12 + jnp.where(xv >= f12, ONE, ZERO)
                    a2 = a2 + jnp.where(xv >= f2, ONE, ZERO)
                return (a1, a12, a2)

            a1, a12, a2 = lax.fori_loop(0, NVEC // UNROLL, count,
                                        (ZERO, ZERO, ZERO))
            n1 = lane_sum(a1)
            n12 = lane_sum(a12)
            n2 = lane_sum(a2)
            k1 = n1 >= K
            k12 = n12 >= K
            k2 = n2 >= K
            new_p = jnp.where(
                k1, jnp.where(k12, c12, c1),
                jnp.where(k2, c2, prefix_u))
            new_c = jnp.where(
                k1, jnp.where(k12, n12, n1),
                jnp.where(k2, n2, cnt_p))
            return (new_p, new_c)

        prefix_u, cnt_p = lax.fori_loop(
            0, S1 // 2, dbl_step, (jnp.int32(0), jnp.int32(0)))

        f_lo = _ordered_bits_to_f32(prefix_u)
        f_hi = _ordered_bits_to_f32(prefix_u + jnp.int32(1 << (32 - S1)))

        def compact1(j, off):
            for u in range(4):
                xv = x_v[r, pl.ds((j * 4 + u) * L, L)]
                m = (xv >= f_lo) & jnp.logical_not(xv >= f_hi)
                plsc.store_compressed(cand_a.at[pl.ds(off, L)], xv,
                                      mask=m)
                off = off + popcnt(m)
            return off

        n_w = lax.fori_loop(0, NVEC // 4, compact1, jnp.int32(0))
        for u in range(4):
            cand_a[pl.ds(n_w + u * L, L)] = NEG_INF
        above = cnt_p - n_w

        trip4 = (n_w + (4 * L - 1)) // (4 * L)

        def cstep2(b, carry):
            prefix_u, cnt_p = carry
            cand_u = prefix_u | lax.shift_left(jnp.int32(1),
                                               31 - S1 - b)
            cand_f = _ordered_bits_to_f32(cand_u)

            def count(j, accs):
                a0, a1, a2, a3 = accs
                base4 = j * (4 * L)
                a0 = a0 + jnp.where(cand_a[pl.ds(base4, L)] >= cand_f,
                                    ONE, ZERO)
                a1 = a1 + jnp.where(
                    cand_a[pl.ds(base4 + L, L)] >= cand_f, ONE, ZERO)
                a2 = a2 + jnp.where(
                    cand_a[pl.ds(base4 + 2 * L, L)] >= cand_f, ONE, ZERO)
                a3 = a3 + jnp.where(
                    cand_a[pl.ds(base4 + 3 * L, L)] >= cand_f, ONE, ZERO)
                return (a0, a1, a2, a3)

            accs = lax.fori_loop(0, trip4, count,
                                 (ZERO, ZERO, ZERO, ZERO))
            cnt = above + lane_sum(accs[0] + accs[1] + accs[2] + accs[3])
            keep = cnt >= K
            return (jnp.where(keep, cand_u, prefix_u),
                    jnp.where(keep, cnt, cnt_p))

        prefix_u, cnt_p = lax.fori_loop(0, S2, cstep2,
                                        (prefix_u, cnt_p))

        f_lo = _ordered_bits_to_f32(prefix_u)
        f_hi = _ordered_bits_to_f32(prefix_u +
                                    jnp.int32(1 << (32 - S1 - S2)))

        def compact2(j, off):
            for u in range(4):
                xv = cand_a[pl.ds((j * 4 + u) * L, L)]
                m = (xv >= f_lo) & jnp.logical_not(xv >= f_hi)
                plsc.store_compressed(cand_b.at[pl.ds(off, L)], xv,
                                      mask=m)
                off = off + popcnt(m)
            return off

        n_w2 = lax.fori_loop(0, trip4, compact2, jnp.int32(0))
        cand_b[pl.ds(n_w2, L)] = NEG_INF
        above2 = cnt_p - n_w2

        trip3 = (n_w2 + (L - 1)) // L

        def cstep3(b, carry):
            prefix_u, cnt_p = carry
            cand_u = prefix_u | lax.shift_left(jnp.int32(1), S3 - 1 - b)
            cand_f = _ordered_bits_to_f32(cand_u)

            def count(j, acc):
                return acc + jnp.where(
                    cand_b[pl.ds(j * L, L)] >= cand_f, ONE, ZERO)

            acc = lax.fori_loop(0, trip3, count, ZERO)
            cnt = above2 + lane_sum(acc)
            keep = cnt >= K
            return (jnp.where(keep, cand_u, prefix_u),
                    jnp.where(keep, cnt, cnt_p))

        prefix_u, cnt_p = lax.fori_loop(0, S3, cstep3,
                                        (prefix_u, cnt_p))

        thr_f = _ordered_bits_to_f32(prefix_u)

        def mask_pass(j, carry):
            for u in range(UNROLL):
                sl = pl.ds((j * UNROLL + u) * L, L)
                xv = x_v[r, sl]
                x_v[r, sl] = jnp.where(xv >= thr_f, xv, jnp.float32(0.0))
            return carry

        lax.fori_loop(0, NVEC // UNROLL, mask_pass, jnp.int32(0))

    pltpu.sync_copy(x_v, out_hbm.at[pl.ds(base, ROWS_PER_W)])


@jax.jit
def kernel(x):
    mesh = plsc.VectorSubcoreMesh(
        core_axis_name="c", subcore_axis_name="s",
        num_cores=NC, num_subcores=NS)
    f = pl.kernel(
        _body,
        out_type=jax.ShapeDtypeStruct((ROWS, N), jnp.float32),
        mesh=mesh,
        compiler_params=pltpu.CompilerParams(needs_layout_passes=False),
        scratch_types=[
            pltpu.VMEM((ROWS_PER_W, N), jnp.float32),
            pltpu.VMEM((N + PAD,), jnp.float32),
            pltpu.VMEM((N + PAD,), jnp.float32),
            pltpu.SemaphoreType.DMA,
        ],
    )
    return f(x)


# speedup vs baseline: 1.2834x; 1.2834x over previous
import jax
import jax.numpy as jnp
from jax import lax
from jax.experimental import pallas as pl
from jax.experimental.pallas import tpu as pltpu
from jax.experimental.pallas import tpu_sc as plsc

ROWS, N = 64, 8192
K = int(0.05 * N)
NC, NS, L = 2, 16, 16
NW = NC * NS
ROWS_PER_W = ROWS // NW
NVEC = N // L
INT_MIN = -2147483648

S1 = 10
S2 = 8
S3 = 32 - S1 - S2
PAD = 4 * L


def _ordered_bits_to_f32(cand_u):
    bits = jnp.where(cand_u < 0, cand_u ^ INT_MIN, ~cand_u)
    return lax.bitcast_convert_type(bits, jnp.float32)


def _body(x_hbm, out_hbm, x_v, cand_a, cand_b, sem):
    wid = lax.axis_index("s") * NC + lax.axis_index("c")
    base = wid * ROWS_PER_W
    pltpu.sync_copy(x_hbm.at[pl.ds(base, ROWS_PER_W)], x_v)

    UNROLL = 8
    ONE = jnp.full((L,), 1, jnp.int32)
    ZERO = jnp.full((L,), 0, jnp.int32)
    NEG_INF = jnp.full((L,), -jnp.inf, jnp.float32)

    def lane_sum(acc):
        return jnp.sum(acc)

    def popcnt(m):
        return plsc.all_reduce_population_count(m)[0]

    for r in range(ROWS_PER_W):
        def bit_step(b, carry):
            prefix_u, cnt_p = carry
            cand_u = prefix_u | lax.shift_left(jnp.int32(1), 31 - b)
            cand_f = _ordered_bits_to_f32(cand_u)

            def count(j, accs):
                new = []
                for u in range(UNROLL):
                    xv = x_v[r, pl.ds((j * UNROLL + u) * L, L)]
                    new.append(accs[u] +
                               jnp.where(xv >= cand_f, ONE, ZERO))
                return tuple(new)

            accs = lax.fori_loop(0, NVEC // UNROLL, count,
                                 tuple(ZERO for _ in range(UNROLL)))
            acc = accs[0]
            for u in range(1, UNROLL):
                acc = acc + accs[u]
            cnt = lane_sum(acc)
            keep = cnt >= K
            return (jnp.where(keep, cand_u, prefix_u),
                    jnp.where(keep, cnt, cnt_p))

        prefix_u, cnt_p = lax.fori_loop(
            0, S1, bit_step, (jnp.int32(0), jnp.int32(0)))

        f_lo = _ordered_bits_to_f32(prefix_u)
        f_hi = _ordered_bits_to_f32(prefix_u + jnp.int32(1 << (32 - S1)))

        def compact1(j, off):
            for u in range(4):
                xv = x_v[r, pl.ds((j * 4 + u) * L, L)]
                m = (xv >= f_lo) & jnp.logical_not(xv >= f_hi)
                plsc.store_compressed(cand_a.at[pl.ds(off, L)], xv,
                                      mask=m)
                off = off + popcnt(m)
            return off

        n_w = lax.fori_loop(0, NVEC // 4, compact1, jnp.int32(0))
        for u in range(4):
            cand_a[pl.ds(n_w + u * L, L)] = NEG_INF
        above = cnt_p - n_w

        trip4 = (n_w + (4 * L - 1)) // (4 * L)

        def cstep2(b, carry):
            prefix_u, cnt_p = carry
            cand_u = prefix_u | lax.shift_left(jnp.int32(1),
                                               31 - S1 - b)
            cand_f = _ordered_bits_to_f32(cand_u)

            def count(j, accs):
                a0, a1, a2, a3 = accs
                base4 = j * (4 * L)
                a0 = a0 + jnp.where(cand_a[pl.ds(base4, L)] >= cand_f,
                                    ONE, ZERO)
                a1 = a1 + jnp.where(
                    cand_a[pl.ds(base4 + L, L)] >= cand_f, ONE, ZERO)
                a2 = a2 + jnp.where(
                    cand_a[pl.ds(base4 + 2 * L, L)] >= cand_f, ONE, ZERO)
                a3 = a3 + jnp.where(
                    cand_a[pl.ds(base4 + 3 * L, L)] >= cand_f, ONE, ZERO)
                return (a0, a1, a2, a3)

            accs = lax.fori_loop(0, trip4, count,
                                 (ZERO, ZERO, ZERO, ZERO))
            cnt = above + lane_sum(accs[0] + accs[1] + accs[2] + accs[3])
            keep = cnt >= K
            return (jnp.where(keep, cand_u, prefix_u),
                    jnp.where(keep, cnt, cnt_p))

        prefix_u, cnt_p = lax.fori_loop(0, S2, cstep2,
                                        (prefix_u, cnt_p))

        f_lo = _ordered_bits_to_f32(prefix_u)
        f_hi = _ordered_bits_to_f32(prefix_u +
                                    jnp.int32(1 << (32 - S1 - S2)))

        def compact2(j, off):
            for u in range(4):
                xv = cand_a[pl.ds((j * 4 + u) * L, L)]
                m = (xv >= f_lo) & jnp.logical_not(xv >= f_hi)
                plsc.store_compressed(cand_b.at[pl.ds(off, L)], xv,
                                      mask=m)
                off = off + popcnt(m)
            return off

        n_w2 = lax.fori_loop(0, trip4, compact2, jnp.int32(0))
        cand_b[pl.ds(n_w2, L)] = NEG_INF
        above2 = cnt_p - n_w2

        trip3 = (n_w2 + (L - 1)) // L

        def cstep3(b, carry):
            prefix_u, cnt_p = carry
            cand_u = prefix_u | lax.shift_left(jnp.int32(1), S3 - 1 - b)
            cand_f = _ordered_bits_to_f32(cand_u)

            def count(j, acc):
                return acc + jnp.where(
                    cand_b[pl.ds(j * L, L)] >= cand_f, ONE, ZERO)

            acc = lax.fori_loop(0, trip3, count, ZERO)
            cnt = above2 + lane_sum(acc)
            keep = cnt >= K
            return (jnp.where(keep, cand_u, prefix_u),
                    jnp.where(keep, cnt, cnt_p))

        prefix_u, cnt_p = lax.fori_loop(0, S3, cstep3,
                                        (prefix_u, cnt_p))

        thr_f = _ordered_bits_to_f32(prefix_u)

        def mask_pass(j, carry):
            for u in range(UNROLL):
                sl = pl.ds((j * UNROLL + u) * L, L)
                xv = x_v[r, sl]
                x_v[r, sl] = jnp.where(xv >= thr_f, xv, jnp.float32(0.0))
            return carry

        lax.fori_loop(0, NVEC // UNROLL, mask_pass, jnp.int32(0))

    pltpu.sync_copy(x_v, out_hbm.at[pl.ds(base, ROWS_PER_W)])


@jax.jit
def kernel(x):
    mesh = plsc.VectorSubcoreMesh(
        core_axis_name="c", subcore_axis_name="s",
        num_cores=NC, num_subcores=NS)
    f = pl.kernel(
        _body,
        out_type=jax.ShapeDtypeStruct((ROWS, N), jnp.float32),
        mesh=mesh,
        compiler_params=pltpu.CompilerParams(needs_layout_passes=False),
        scratch_types=[
            pltpu.VMEM((ROWS_PER_W, N), jnp.float32),
            pltpu.VMEM((N + PAD,), jnp.float32),
            pltpu.VMEM((N + PAD,), jnp.float32),
            pltpu.SemaphoreType.DMA,
        ],
    )
    return f(x)


# speedup vs baseline: 1.2965x; 1.0103x over previous
import jax
import jax.numpy as jnp
from jax import lax
from jax.experimental import pallas as pl
from jax.experimental.pallas import tpu as pltpu
from jax.experimental.pallas import tpu_sc as plsc

ROWS, N = 64, 8192
K = int(0.05 * N)
NC, NS, L = 2, 16, 16
NW = NC * NS
ROWS_PER_W = ROWS // NW
NVEC = N // L
INT_MIN = -2147483648

S1 = 10
S2 = 32 - S1
PAD = 4 * L


def _ordered_bits_to_f32(cand_u):
    bits = jnp.where(cand_u < 0, cand_u ^ INT_MIN, ~cand_u)
    return lax.bitcast_convert_type(bits, jnp.float32)


def _body(x_hbm, out_hbm, x_v, cand_a, sem):
    wid = lax.axis_index("s") * NC + lax.axis_index("c")
    base = wid * ROWS_PER_W
    pltpu.sync_copy(x_hbm.at[pl.ds(base, ROWS_PER_W)], x_v)

    UNROLL = 8
    ONE = jnp.full((L,), 1, jnp.int32)
    ZERO = jnp.full((L,), 0, jnp.int32)
    NEG_INF = jnp.full((L,), -jnp.inf, jnp.float32)

    def lane_sum(acc):
        return jnp.sum(acc)

    def popcnt(m):
        return plsc.all_reduce_population_count(m)[0]

    for r in range(ROWS_PER_W):
        def bit_step(b, carry):
            prefix_u, cnt_p = carry
            cand_u = prefix_u | lax.shift_left(jnp.int32(1), 31 - b)
            cand_f = _ordered_bits_to_f32(cand_u)

            def count(j, accs):
                new = []
                for u in range(UNROLL):
                    xv = x_v[r, pl.ds((j * UNROLL + u) * L, L)]
                    new.append(accs[u] +
                               jnp.where(xv >= cand_f, ONE, ZERO))
                return tuple(new)

            accs = lax.fori_loop(0, NVEC // UNROLL, count,
                                 tuple(ZERO for _ in range(UNROLL)))
            acc = accs[0]
            for u in range(1, UNROLL):
                acc = acc + accs[u]
            cnt = lane_sum(acc)
            keep = cnt >= K
            return (jnp.where(keep, cand_u, prefix_u),
                    jnp.where(keep, cnt, cnt_p))

        prefix_u, cnt_p = lax.fori_loop(
            0, S1, bit_step, (jnp.int32(0), jnp.int32(0)))

        f_lo = _ordered_bits_to_f32(prefix_u)
        f_hi = _ordered_bits_to_f32(prefix_u + jnp.int32(1 << (32 - S1)))

        def compact1(j, off):
            for u in range(8):
                xv = x_v[r, pl.ds((j * 8 + u) * L, L)]
                m = (xv >= f_lo) & jnp.logical_not(xv >= f_hi)
                plsc.store_compressed(cand_a.at[pl.ds(off, L)], xv,
                                      mask=m)
                off = off + popcnt(m)
            return off

        n_w = lax.fori_loop(0, NVEC // 8, compact1, jnp.int32(0))
        for u in range(4):
            cand_a[pl.ds(n_w + u * L, L)] = NEG_INF
        above = cnt_p - n_w

        trip4 = (n_w + (4 * L - 1)) // (4 * L)

        def cstep2(b, carry):
            prefix_u, cnt_p = carry
            cand_u = prefix_u | lax.shift_left(jnp.int32(1),
                                               31 - S1 - b)
            cand_f = _ordered_bits_to_f32(cand_u)

            def count(j, accs):
                a0, a1, a2, a3 = accs
                base4 = j * (4 * L)
                a0 = a0 + jnp.where(cand_a[pl.ds(base4, L)] >= cand_f,
                                    ONE, ZERO)
                a1 = a1 + jnp.where(
                    cand_a[pl.ds(base4 + L, L)] >= cand_f, ONE, ZERO)
                a2 = a2 + jnp.where(
                    cand_a[pl.ds(base4 + 2 * L, L)] >= cand_f, ONE, ZERO)
                a3 = a3 + jnp.where(
                    cand_a[pl.ds(base4 + 3 * L, L)] >= cand_f, ONE, ZERO)
                return (a0, a1, a2, a3)

            accs = lax.fori_loop(0, trip4, count,
                                 (ZERO, ZERO, ZERO, ZERO))
            cnt = above + lane_sum(accs[0] + accs[1] + accs[2] + accs[3])
            keep = cnt >= K
            return (jnp.where(keep, cand_u, prefix_u),
                    jnp.where(keep, cnt, cnt_p))

        prefix_u, cnt_p = lax.fori_loop(0, S2, cstep2,
                                        (prefix_u, cnt_p))

        thr_f = _ordered_bits_to_f32(prefix_u)

        def mask_pass(j, carry):
            for u in range(UNROLL):
                sl = pl.ds((j * UNROLL + u) * L, L)
                xv = x_v[r, sl]
                x_v[r, sl] = jnp.where(xv >= thr_f, xv, jnp.float32(0.0))
            return carry

        lax.fori_loop(0, NVEC // UNROLL, mask_pass, jnp.int32(0))

    pltpu.sync_copy(x_v, out_hbm.at[pl.ds(base, ROWS_PER_W)])


@jax.jit
def kernel(x):
    mesh = plsc.VectorSubcoreMesh(
        core_axis_name="c", subcore_axis_name="s",
        num_cores=NC, num_subcores=NS)
    f = pl.kernel(
        _body,
        out_type=jax.ShapeDtypeStruct((ROWS, N), jnp.float32),
        mesh=mesh,
        compiler_params=pltpu.CompilerParams(needs_layout_passes=False),
        scratch_types=[
            pltpu.VMEM((ROWS_PER_W, N), jnp.float32),
            pltpu.VMEM((N + PAD,), jnp.float32),
            pltpu.SemaphoreType.DMA,
        ],
    )
    return f(x)


# speedup vs baseline: 1.5812x; 1.2195x over previous
import jax
import jax.numpy as jnp
from jax import lax
from jax.experimental import pallas as pl
from jax.experimental.pallas import tpu as pltpu
from jax.experimental.pallas import tpu_sc as plsc

ROWS, N = 64, 8192
K = int(0.05 * N)
NC, NS, L = 2, 16, 16
NW = NC * NS
ROWS_SC = 32
ROWS_TC = ROWS - ROWS_SC
ROWS_PER_W = ROWS_SC // NW
NVEC = N // L
INT_MIN = -2147483648

S1 = 10
S2 = 32 - S1
PAD = 4 * L


def _ordered_bits_to_f32(cand_u):
    bits = jnp.where(cand_u < 0, cand_u ^ INT_MIN, ~cand_u)
    return lax.bitcast_convert_type(bits, jnp.float32)


def _body(x_hbm, out_hbm, x_v, cand_a, sem):
    wid = lax.axis_index("s") * NC + lax.axis_index("c")
    base = wid * ROWS_PER_W
    pltpu.sync_copy(x_hbm.at[pl.ds(base, ROWS_PER_W)], x_v)

    UNROLL = 8
    ONE = jnp.full((L,), 1, jnp.int32)
    ZERO = jnp.full((L,), 0, jnp.int32)
    NEG_INF = jnp.full((L,), -jnp.inf, jnp.float32)

    def lane_sum(acc):
        return jnp.sum(acc)

    def popcnt(m):
        return plsc.all_reduce_population_count(m)[0]

    for r in range(ROWS_PER_W):
        def bit_step(b, carry):
            prefix_u, cnt_p = carry
            cand_u = prefix_u | lax.shift_left(jnp.int32(1), 31 - b)
            cand_f = _ordered_bits_to_f32(cand_u)

            def count(j, accs):
                new = []
                for u in range(UNROLL):
                    xv = x_v[r, pl.ds((j * UNROLL + u) * L, L)]
                    new.append(accs[u] +
                               jnp.where(xv >= cand_f, ONE, ZERO))
                return tuple(new)

            accs = lax.fori_loop(0, NVEC // UNROLL, count,
                                 tuple(ZERO for _ in range(UNROLL)))
            acc = accs[0]
            for u in range(1, UNROLL):
                acc = acc + accs[u]
            cnt = lane_sum(acc)
            keep = cnt >= K
            return (jnp.where(keep, cand_u, prefix_u),
                    jnp.where(keep, cnt, cnt_p))

        prefix_u, cnt_p = lax.fori_loop(
            0, S1, bit_step, (jnp.int32(0), jnp.int32(0)))

        f_lo = _ordered_bits_to_f32(prefix_u)
        f_hi = _ordered_bits_to_f32(prefix_u + jnp.int32(1 << (32 - S1)))

        def compact1(j, off):
            for u in range(8):
                xv = x_v[r, pl.ds((j * 8 + u) * L, L)]
                m = (xv >= f_lo) & jnp.logical_not(xv >= f_hi)
                plsc.store_compressed(cand_a.at[pl.ds(off, L)], xv,
                                      mask=m)
                off = off + popcnt(m)
            return off

        n_w = lax.fori_loop(0, NVEC // 8, compact1, jnp.int32(0))
        for u in range(4):
            cand_a[pl.ds(n_w + u * L, L)] = NEG_INF
        above = cnt_p - n_w

        trip4 = (n_w + (4 * L - 1)) // (4 * L)

        def cstep2(b, carry):
            prefix_u, cnt_p = carry
            cand_u = prefix_u | lax.shift_left(jnp.int32(1),
                                               31 - S1 - b)
            cand_f = _ordered_bits_to_f32(cand_u)

            def count(j, accs):
                a0, a1, a2, a3 = accs
                base4 = j * (4 * L)
                a0 = a0 + jnp.where(cand_a[pl.ds(base4, L)] >= cand_f,
                                    ONE, ZERO)
                a1 = a1 + jnp.where(
                    cand_a[pl.ds(base4 + L, L)] >= cand_f, ONE, ZERO)
                a2 = a2 + jnp.where(
                    cand_a[pl.ds(base4 + 2 * L, L)] >= cand_f, ONE, ZERO)
                a3 = a3 + jnp.where(
                    cand_a[pl.ds(base4 + 3 * L, L)] >= cand_f, ONE, ZERO)
                return (a0, a1, a2, a3)

            accs = lax.fori_loop(0, trip4, count,
                                 (ZERO, ZERO, ZERO, ZERO))
            cnt = above + lane_sum(accs[0] + accs[1] + accs[2] + accs[3])
            keep = cnt >= K
            return (jnp.where(keep, cand_u, prefix_u),
                    jnp.where(keep, cnt, cnt_p))

        prefix_u, cnt_p = lax.fori_loop(0, S2, cstep2,
                                        (prefix_u, cnt_p))

        thr_f = _ordered_bits_to_f32(prefix_u)

        def mask_pass(j, carry):
            for u in range(UNROLL):
                sl = pl.ds((j * UNROLL + u) * L, L)
                xv = x_v[r, sl]
                x_v[r, sl] = jnp.where(xv >= thr_f, xv, jnp.float32(0.0))
            return carry

        lax.fori_loop(0, NVEC // UNROLL, mask_pass, jnp.int32(0))

    pltpu.sync_copy(x_v, out_hbm.at[pl.ds(base, ROWS_PER_W)])


def _tc_body(x_ref, o_ref):
    x = x_ref[...]
    xi = lax.bitcast_convert_type(x, jnp.int32)
    skey = jnp.where(xi >= 0, xi, xi ^ jnp.int32(0x7FFFFFFF))
    skey = jnp.where(xi == jnp.int32(INT_MIN), jnp.int32(0), skey)

    def bit_step(b, prefix_u):
        cand_u = prefix_u | lax.shift_left(jnp.int32(1), 31 - b)
        cand_s = cand_u ^ jnp.int32(INT_MIN)
        cnt = jnp.sum((skey >= cand_s).astype(jnp.int32), axis=1,
                      keepdims=True)
        return jnp.where(cnt >= K, cand_u, prefix_u)

    prefix_u = lax.fori_loop(
        0, 32, bit_step, jnp.zeros((ROWS_TC, 1), jnp.int32))
    thr_s = prefix_u ^ jnp.int32(INT_MIN)
    o_ref[...] = jnp.where(skey >= thr_s, x, jnp.float32(0.0))


@jax.jit
def kernel(x):
    mesh = plsc.VectorSubcoreMesh(
        core_axis_name="c", subcore_axis_name="s",
        num_cores=NC, num_subcores=NS)
    f_sc = pl.kernel(
        _body,
        out_type=jax.ShapeDtypeStruct((ROWS_SC, N), jnp.float32),
        mesh=mesh,
        compiler_params=pltpu.CompilerParams(needs_layout_passes=False),
        scratch_types=[
            pltpu.VMEM((ROWS_PER_W, N), jnp.float32),
            pltpu.VMEM((N + PAD,), jnp.float32),
            pltpu.SemaphoreType.DMA,
        ],
    )
    f_tc = pl.pallas_call(
        _tc_body,
        out_shape=jax.ShapeDtypeStruct((ROWS_TC, N), jnp.float32),
    )
    out_sc = f_sc(x[:ROWS_SC])
    out_tc = f_tc(x[ROWS_SC:])
    return jnp.concatenate([out_sc, out_tc], axis=0)
